# Initial kernel scaffold; baseline (speedup 1.0000x reference)
import functools

import jax
import jax.numpy as jnp
from jax import lax
from jax.experimental import pallas as pl
from jax.experimental.pallas import tpu as pltpu

N_Q = 1024
N_L = 16
D = 384
H = 64
N_ROLES = 1024
QB = 16
CHUNK = 128
R = QB * N_L


def _body(q_ref, t_ref, r_ref, l_ref, rembT_ref, wq_ref, bq_ref,
          wrT_ref, brT_ref, u_ref, act_ref,
          sel_ref, lp_out_ref, rt_ref, lr_ref, lpart_ref):
    @pl.when(pl.program_id(0) == 0)
    def _init():
        rt_un = (jnp.dot(wrT_ref[...], rembT_ref[...],
                         preferred_element_type=jnp.float32)
                 + brT_ref[...])
        n = jnp.sqrt(jnp.sum(rt_un * rt_un, axis=0, keepdims=True))
        rt_ref[...] = rt_un * (1.0 / jnp.maximum(n, 1e-12))
        lp = jnp.dot(l_ref[...], wq_ref[2 * D:3 * D, :],
                     preferred_element_type=jnp.float32)
        lpart_ref[...] = lp
        lr_ref[...] = jnp.dot(lp, rt_ref[...],
                              preferred_element_type=jnp.float32)

    s = (jnp.dot(q_ref[...], wq_ref[0:D, :],
                 preferred_element_type=jnp.float32)
         + jnp.dot(t_ref[...], wq_ref[D:2 * D, :],
                   preferred_element_type=jnp.float32)
         + jnp.dot(r_ref[...], wq_ref[3 * D:4 * D, :],
                   preferred_element_type=jnp.float32)
         + bq_ref[...])

    lp = lpart_ref[...]
    e3 = s[:, None, :] + lp[None, :, :]
    n2 = jnp.sum(e3 * e3, axis=-1)
    inv = 1.0 / jnp.maximum(jnp.sqrt(n2), 1e-12)

    sr = jnp.dot(s, rt_ref[...], preferred_element_type=jnp.float32)
    x3 = (sr[:, None, :] + lr_ref[...][None, :, :]) * inv[:, :, None]
    m = jnp.max(x3, axis=-1, keepdims=True)
    p3 = jnp.exp(x3 - m)
    z3 = jnp.sum(p3, axis=-1)

    e = p3.reshape(R, N_ROLES)
    thr = (u_ref[...] * z3).reshape(R, 1)

    row = lax.broadcasted_iota(jnp.int32, (CHUNK, CHUNK), 0)
    col = lax.broadcasted_iota(jnp.int32, (CHUNK, CHUNK), 1)
    tri = (row <= col).astype(jnp.float32)
    cnt = jnp.zeros((R, 1), jnp.int32)
    carry = jnp.zeros((R, 1), jnp.float32)
    for c in range(N_ROLES // CHUNK):
        f = jnp.dot(e[:, c * CHUNK:(c + 1) * CHUNK], tri,
                    preferred_element_type=jnp.float32)
        csum = f + carry
        cnt = cnt + jnp.sum((csum <= thr).astype(jnp.int32), axis=-1,
                            keepdims=True)
        carry = csum[:, CHUNK - 1:CHUNK]

    sel = jnp.where(cnt >= N_ROLES, 0, cnt)

    iota = lax.broadcasted_iota(jnp.int32, (R, N_ROLES), 1)
    sel_e = jnp.sum(jnp.where(iota == sel, e, 0.0), axis=-1, keepdims=True)

    logterm = jnp.log(sel_e) - jnp.log(z3.reshape(R, 1))
    lt = logterm.reshape(QB, N_L) * act_ref[...]
    sel_ref[...] = sel.reshape(QB, N_L)
    lp_out_ref[...] = jnp.sum(lt, axis=-1, keepdims=True)


@jax.jit
def _run(queries, tasks, reasonings, llms_embedding, role_embT, W_qtlr,
         b_qtlr2, W_roleT, b_roleT, rand_u2, active):
    grid = (N_Q // QB,)
    blk = lambda shape: pl.BlockSpec(
        shape, lambda i: (i,) + (0,) * (len(shape) - 1))
    rep = lambda shape: pl.BlockSpec(shape, lambda i: (0,) * len(shape))
    out = pl.pallas_call(
        _body,
        grid=grid,
        in_specs=[
            blk((QB, D)), blk((QB, D)), blk((QB, D)),
            rep((N_L, D)), rep((D, N_ROLES)), rep((4 * D, H)),
            rep((1, H)), rep((H, D)), rep((H, 1)),
            blk((QB, N_L)), blk((QB, N_L)),
        ],
        out_specs=[blk((QB, N_L)), blk((QB, 1))],
        out_shape=[
            jax.ShapeDtypeStruct((N_Q, N_L), jnp.int32),
            jax.ShapeDtypeStruct((N_Q, 1), jnp.float32),
        ],
        scratch_shapes=[
            pltpu.VMEM((H, N_ROLES), jnp.float32),
            pltpu.VMEM((N_L, N_ROLES), jnp.float32),
            pltpu.VMEM((N_L, H), jnp.float32),
        ],
    )(queries, tasks, reasonings, llms_embedding, role_embT, W_qtlr,
      b_qtlr2, W_roleT, b_roleT, rand_u2, active)
    return out[0], out[1]


def kernel(queries, tasks, llms_embedding, llms_num, reasonings, role_emb,
           W_qtlr, b_qtlr, W_role, b_role, rand_u):
    role_embT = role_emb.T
    W_roleT = W_role.T
    b_qtlr2 = b_qtlr.reshape(1, H)
    b_roleT = b_role.reshape(H, 1)
    rand_u2 = rand_u.reshape(N_Q, N_L)
    active = (llms_num > 0).astype(jnp.float32)
    return _run(queries, tasks, reasonings, llms_embedding, role_embT,
                W_qtlr, b_qtlr2, W_roleT, b_roleT, rand_u2, active)


# speedup vs baseline: 1.4398x; 1.4398x over previous
import jax
import jax.numpy as jnp
from jax import lax
from jax.experimental import pallas as pl
from jax.experimental.pallas import tpu as pltpu

N_Q = 1024
N_L = 16
D = 384
H = 64
N_ROLES = 1024
QB = 16
CHUNK = 128
R = QB * N_L


def _body(q_ref, t_ref, r_ref, l_ref, rembT_ref, wq_ref, bq_ref,
          wrT_ref, brT_ref, u_ref, act_ref,
          sel_ref, lp_out_ref, rt_ref, lr_ref, lpart_ref):
    f32 = jnp.float32
    @pl.when(pl.program_id(0) == 0)
    def _init():
        rt_un = (jnp.dot(wrT_ref[...], rembT_ref[...],
                         preferred_element_type=f32)
                 + brT_ref[...])
        n = jnp.sqrt(jnp.sum(rt_un * rt_un, axis=0, keepdims=True))
        rt_ref[...] = rt_un * (1.0 / jnp.maximum(n, 1e-12))
        lp = jnp.dot(l_ref[...], wq_ref[2 * D:3 * D, :],
                     preferred_element_type=f32)
        lpart_ref[...] = lp
        lr_ref[...] = jnp.dot(lp, rt_ref[...], preferred_element_type=f32)

    s = (jnp.dot(q_ref[...], wq_ref[0:D, :], preferred_element_type=f32)
         + jnp.dot(t_ref[...], wq_ref[D:2 * D, :],
                   preferred_element_type=f32)
         + jnp.dot(r_ref[...], wq_ref[3 * D:4 * D, :],
                   preferred_element_type=f32)
         + bq_ref[...])

    sr = jnp.dot(s, rt_ref[...], preferred_element_type=f32)

    rr = lax.broadcasted_iota(jnp.int32, (R, QB + N_L), 0)
    cc = lax.broadcasted_iota(jnp.int32, (R, QB + N_L), 1)
    g2 = jnp.where(cc < QB, (cc == rr // N_L).astype(f32),
                   (cc - QB == rr % N_L).astype(f32))

    slp = jnp.concatenate([s, lpart_ref[...]], axis=0)
    e2 = jnp.dot(g2, slp, preferred_element_type=f32)
    n2 = jnp.sum(e2 * e2, axis=-1, keepdims=True)
    inv = 1.0 / jnp.maximum(jnp.sqrt(n2), 1e-12)

    srlr = jnp.concatenate([sr, lr_ref[...]], axis=0)
    x = jnp.dot(g2, srlr, preferred_element_type=f32) * inv

    m = jnp.max(x, axis=-1, keepdims=True)
    e = jnp.exp(x - m)
    z = jnp.sum(e, axis=-1, keepdims=True)
    thr = u_ref[...] * z

    row = lax.broadcasted_iota(jnp.int32, (CHUNK, CHUNK), 0)
    col = lax.broadcasted_iota(jnp.int32, (CHUNK, CHUNK), 1)
    tri = (row <= col).astype(f32)
    cnt = jnp.zeros((R, 1), jnp.int32)
    carry = jnp.zeros((R, 1), f32)
    for c in range(N_ROLES // CHUNK):
        f = jnp.dot(e[:, c * CHUNK:(c + 1) * CHUNK], tri,
                    preferred_element_type=f32)
        csum = f + carry
        cnt = cnt + jnp.sum((csum <= thr).astype(jnp.int32), axis=-1,
                            keepdims=True)
        carry = csum[:, CHUNK - 1:CHUNK]

    sel = jnp.where(cnt >= N_ROLES, 0, cnt)

    iota = lax.broadcasted_iota(jnp.int32, (R, N_ROLES), 1)
    sel_e = jnp.sum(jnp.where(iota == sel, e, 0.0), axis=-1, keepdims=True)

    logterm = (jnp.log(sel_e) - jnp.log(z)) * act_ref[...]

    qq = lax.broadcasted_iota(jnp.int32, (QB, R), 0)
    rc = lax.broadcasted_iota(jnp.int32, (QB, R), 1)
    gq = (qq == rc // N_L).astype(f32)
    sel_ref[...] = sel
    lp_out_ref[...] = jnp.dot(gq, logterm, preferred_element_type=f32)


@jax.jit
def _run(queries, tasks, reasonings, llms_embedding, role_embT, W_qtlr,
         b_qtlr2, W_roleT, b_roleT, rand_u2, active):
    grid = (N_Q // QB,)
    blk = lambda shape: pl.BlockSpec(
        shape, lambda i: (i,) + (0,) * (len(shape) - 1))
    rep = lambda shape: pl.BlockSpec(shape, lambda i: (0,) * len(shape))
    out = pl.pallas_call(
        _body,
        grid=grid,
        in_specs=[
            blk((QB, D)), blk((QB, D)), blk((QB, D)),
            rep((N_L, D)), rep((D, N_ROLES)), rep((4 * D, H)),
            rep((1, H)), rep((H, D)), rep((H, 1)),
            blk((R, 1)), blk((R, 1)),
        ],
        out_specs=[blk((R, 1)), blk((QB, 1))],
        out_shape=[
            jax.ShapeDtypeStruct((N_Q * N_L, 1), jnp.int32),
            jax.ShapeDtypeStruct((N_Q, 1), jnp.float32),
        ],
        scratch_shapes=[
            pltpu.VMEM((H, N_ROLES), jnp.float32),
            pltpu.VMEM((N_L, N_ROLES), jnp.float32),
            pltpu.VMEM((N_L, H), jnp.float32),
        ],
    )(queries, tasks, reasonings, llms_embedding, role_embT, W_qtlr,
      b_qtlr2, W_roleT, b_roleT, rand_u2, active)
    return out[0].reshape(N_Q, N_L), out[1]


def kernel(queries, tasks, llms_embedding, llms_num, reasonings, role_emb,
           W_qtlr, b_qtlr, W_role, b_role, rand_u):
    role_embT = role_emb.T
    W_roleT = W_role.T
    b_qtlr2 = b_qtlr.reshape(1, H)
    b_roleT = b_role.reshape(H, 1)
    rand_u2 = rand_u.reshape(N_Q * N_L, 1)
    active = (llms_num > 0).astype(jnp.float32).reshape(N_Q * N_L, 1)
    return _run(queries, tasks, reasonings, llms_embedding, role_embT,
                W_qtlr, b_qtlr2, W_roleT, b_roleT, rand_u2, active)


# speedup vs baseline: 2.8162x; 1.9561x over previous
import jax
import jax.numpy as jnp
from jax import lax
from jax.experimental import pallas as pl
from jax.experimental.pallas import tpu as pltpu

N_Q = 1024
N_L = 16
D = 384
H = 64
N_ROLES = 1024
QB = 16
CHUNK = 128
NCH = N_ROLES // CHUNK
R = QB * N_L


def _body(qtr_ref, l_ref, rembT_ref, wqtr_ref, wl_ref, bq_ref,
          wrT_ref, brT_ref, u_ref, act_ref,
          sel_ref, lp_out_ref, rt_ref, lr_ref, lpart_ref):
    f32 = jnp.float32

    @pl.when(pl.program_id(0) == 0)
    def _init():
        rt_un = (jnp.dot(wrT_ref[...], rembT_ref[...],
                         preferred_element_type=f32)
                 + brT_ref[...])
        n = jnp.sqrt(jnp.sum(rt_un * rt_un, axis=0, keepdims=True))
        rt_ref[...] = rt_un * (1.0 / jnp.maximum(n, 1e-12))
        lp = jnp.dot(l_ref[...], wl_ref[...], preferred_element_type=f32)
        lpart_ref[...] = lp
        lr_ref[...] = jnp.dot(lp, rt_ref[...], preferred_element_type=f32)

    s = (jnp.dot(qtr_ref[...], wqtr_ref[...], preferred_element_type=f32)
         + bq_ref[...])

    sr = jnp.dot(s, rt_ref[...], preferred_element_type=f32)

    rr = lax.broadcasted_iota(jnp.int32, (R, QB + N_L), 0)
    cc = lax.broadcasted_iota(jnp.int32, (R, QB + N_L), 1)
    g2 = jnp.where(cc < QB, (cc == rr // N_L).astype(f32),
                   (cc - QB == rr % N_L).astype(f32))

    slp = jnp.concatenate([s, lpart_ref[...]], axis=0)
    e2 = jnp.dot(g2, slp, preferred_element_type=f32)
    n2 = jnp.sum(e2 * e2, axis=-1, keepdims=True)
    inv = 1.0 / jnp.maximum(jnp.sqrt(n2), 1e-12)

    srlr = jnp.concatenate([sr, lr_ref[...]], axis=0)
    e = jnp.exp(jnp.dot(g2, srlr, preferred_element_type=f32) * inv)

    row = lax.broadcasted_iota(jnp.int32, (CHUNK, CHUNK), 0)
    col = lax.broadcasted_iota(jnp.int32, (CHUNK, CHUNK), 1)
    tri = (row <= col).astype(f32)
    fs = [jnp.dot(e[:, c * CHUNK:(c + 1) * CHUNK], tri,
                  preferred_element_type=f32) for c in range(NCH)]
    carries = [jnp.zeros((R, 1), f32)]
    for c in range(NCH):
        carries.append(carries[c] + fs[c][:, CHUNK - 1:CHUNK])
    z = carries[NCH]
    thr = u_ref[...] * z

    cntv = jnp.zeros((R, CHUNK), f32)
    for c in range(NCH):
        cntv = cntv + (fs[c] <= thr - carries[c]).astype(f32)
    ones_c = jnp.ones((CHUNK, 1), f32)
    cnt = jnp.dot(cntv, ones_c, preferred_element_type=f32).astype(jnp.int32)
    sel = jnp.where(cnt >= N_ROLES, 0, cnt)

    iota = lax.broadcasted_iota(jnp.int32, (R, N_ROLES), 1)
    masked = jnp.where(iota == sel, e, 0.0)
    ones_n = jnp.ones((N_ROLES, 1), f32)
    sel_e = jnp.dot(masked, ones_n, preferred_element_type=f32)

    logterm = (jnp.log(sel_e) - jnp.log(z)) * act_ref[...]

    qq = lax.broadcasted_iota(jnp.int32, (QB, R), 0)
    rc = lax.broadcasted_iota(jnp.int32, (QB, R), 1)
    gq = (qq == rc // N_L).astype(f32)
    sel_ref[...] = sel
    lp_out_ref[...] = jnp.dot(gq, logterm, preferred_element_type=f32)


@jax.jit
def _run(qtr, llms_embedding, role_embT, W_qtr, W_l,
         b_qtlr2, W_roleT, b_roleT, rand_u2, active):
    grid = (N_Q // QB,)
    blk = lambda shape: pl.BlockSpec(
        shape, lambda i: (i,) + (0,) * (len(shape) - 1))
    rep = lambda shape: pl.BlockSpec(shape, lambda i: (0,) * len(shape))
    out = pl.pallas_call(
        _body,
        grid=grid,
        in_specs=[
            blk((QB, 3 * D)),
            rep((N_L, D)), rep((D, N_ROLES)), rep((3 * D, H)),
            rep((D, H)), rep((1, H)), rep((H, D)), rep((H, 1)),
            blk((R, 1)), blk((R, 1)),
        ],
        out_specs=[blk((R, 1)), blk((QB, 1))],
        out_shape=[
            jax.ShapeDtypeStruct((N_Q * N_L, 1), jnp.int32),
            jax.ShapeDtypeStruct((N_Q, 1), jnp.float32),
        ],
        scratch_shapes=[
            pltpu.VMEM((H, N_ROLES), jnp.float32),
            pltpu.VMEM((N_L, N_ROLES), jnp.float32),
            pltpu.VMEM((N_L, H), jnp.float32),
        ],
    )(qtr, llms_embedding, role_embT, W_qtr, W_l,
      b_qtlr2, W_roleT, b_roleT, rand_u2, active)
    return out[0].reshape(N_Q, N_L), out[1]


def kernel(queries, tasks, llms_embedding, llms_num, reasonings, role_emb,
           W_qtlr, b_qtlr, W_role, b_role, rand_u):
    qtr = jnp.concatenate([queries, tasks, reasonings], axis=1)
    W_qtr = jnp.concatenate([W_qtlr[0:D], W_qtlr[D:2 * D],
                             W_qtlr[3 * D:4 * D]], axis=0)
    W_l = W_qtlr[2 * D:3 * D]
    role_embT = role_emb.T
    W_roleT = W_role.T
    b_qtlr2 = b_qtlr.reshape(1, H)
    b_roleT = b_role.reshape(H, 1)
    rand_u2 = rand_u.reshape(N_Q * N_L, 1)
    active = (llms_num > 0).astype(jnp.float32).reshape(N_Q * N_L, 1)
    return _run(qtr, llms_embedding, role_embT, W_qtr, W_l,
                b_qtlr2, W_roleT, b_roleT, rand_u2, active)


# speedup vs baseline: 3.3877x; 1.2029x over previous
import jax
import jax.numpy as jnp
from jax import lax
from jax.experimental import pallas as pl
from jax.experimental.pallas import tpu as pltpu

N_Q = 1024
N_L = 16
D = 384
H = 64
N_ROLES = 1024
QB = 32
CHUNK = 128
NCH = N_ROLES // CHUNK
R = QB * N_L


def _body(qtr_ref, l_ref, rembT_ref, wqtr_ref, wl_ref, bq_ref,
          wrT_ref, brT_ref, u_ref, act_ref,
          sel_ref, lp_out_ref, rt_ref, lr_ref, lpart_ref,
          tri_ref, g2_ref, gq_ref):
    f32 = jnp.float32

    @pl.when(pl.program_id(0) == 0)
    def _init():
        row = lax.broadcasted_iota(jnp.int32, (CHUNK, CHUNK), 0)
        col = lax.broadcasted_iota(jnp.int32, (CHUNK, CHUNK), 1)
        tri_ref[...] = (row <= col).astype(f32)
        rr = lax.broadcasted_iota(jnp.int32, (R, QB + N_L), 0)
        cc = lax.broadcasted_iota(jnp.int32, (R, QB + N_L), 1)
        g2_ref[...] = jnp.where(cc < QB, (cc == rr // N_L).astype(f32),
                                (cc - QB == rr % N_L).astype(f32))
        qq = lax.broadcasted_iota(jnp.int32, (QB, R), 0)
        rc = lax.broadcasted_iota(jnp.int32, (QB, R), 1)
        gq_ref[...] = (qq == rc // N_L).astype(f32)
        rt_un = (jnp.dot(wrT_ref[...], rembT_ref[...],
                         preferred_element_type=f32)
                 + brT_ref[...])
        n = jnp.sqrt(jnp.sum(rt_un * rt_un, axis=0, keepdims=True))
        rt_ref[...] = rt_un * (1.0 / jnp.maximum(n, 1e-12))
        lp = jnp.dot(l_ref[...], wl_ref[...], preferred_element_type=f32)
        lpart_ref[...] = lp
        lr_ref[...] = jnp.dot(lp, rt_ref[...], preferred_element_type=f32)

    s = (jnp.dot(qtr_ref[...], wqtr_ref[...], preferred_element_type=f32)
         + bq_ref[...])

    sr = jnp.dot(s, rt_ref[...], preferred_element_type=f32)

    g2 = g2_ref[...]

    slp = jnp.concatenate([s, lpart_ref[...]], axis=0)
    e2 = jnp.dot(g2, slp, preferred_element_type=f32)
    n2 = jnp.sum(e2 * e2, axis=-1, keepdims=True)
    inv = 1.0 / jnp.maximum(jnp.sqrt(n2), 1e-12)

    srlr = jnp.concatenate([sr, lr_ref[...]], axis=0)
    e = jnp.exp(jnp.dot(g2, srlr, preferred_element_type=f32) * inv)

    tri = tri_ref[...]
    fs = [jnp.dot(e[:, c * CHUNK:(c + 1) * CHUNK], tri,
                  preferred_element_type=f32) for c in range(NCH)]
    carries = [jnp.zeros((R, 1), f32)]
    for c in range(NCH):
        carries.append(carries[c] + fs[c][:, CHUNK - 1:CHUNK])
    z = carries[NCH]
    thr = u_ref[...] * z

    cntv = jnp.zeros((R, CHUNK), f32)
    for c in range(NCH):
        cntv = cntv + (fs[c] <= thr - carries[c]).astype(f32)
    ones_c = jnp.ones((CHUNK, 1), f32)
    cnt = jnp.dot(cntv, ones_c, preferred_element_type=f32).astype(jnp.int32)
    sel = jnp.where(cnt >= N_ROLES, 0, cnt)

    iota = lax.broadcasted_iota(jnp.int32, (R, N_ROLES), 1)
    masked = jnp.where(iota == sel, e, 0.0)
    ones_n = jnp.ones((N_ROLES, 1), f32)
    sel_e = jnp.dot(masked, ones_n, preferred_element_type=f32)

    logterm = (jnp.log(sel_e) - jnp.log(z)) * act_ref[...]

    sel_ref[...] = sel
    lp_out_ref[...] = jnp.dot(gq_ref[...], logterm,
                              preferred_element_type=f32)


@jax.jit
def _run(qtr, llms_embedding, role_embT, W_qtr, W_l,
         b_qtlr2, W_roleT, b_roleT, rand_u2, active):
    grid = (N_Q // QB,)
    blk = lambda shape: pl.BlockSpec(
        shape, lambda i: (i,) + (0,) * (len(shape) - 1))
    rep = lambda shape: pl.BlockSpec(shape, lambda i: (0,) * len(shape))
    out = pl.pallas_call(
        _body,
        grid=grid,
        in_specs=[
            blk((QB, 3 * D)),
            rep((N_L, D)), rep((D, N_ROLES)), rep((3 * D, H)),
            rep((D, H)), rep((1, H)), rep((H, D)), rep((H, 1)),
            blk((R, 1)), blk((R, 1)),
        ],
        out_specs=[blk((R, 1)), blk((QB, 1))],
        out_shape=[
            jax.ShapeDtypeStruct((N_Q * N_L, 1), jnp.int32),
            jax.ShapeDtypeStruct((N_Q, 1), jnp.float32),
        ],
        scratch_shapes=[
            pltpu.VMEM((H, N_ROLES), jnp.float32),
            pltpu.VMEM((N_L, N_ROLES), jnp.float32),
            pltpu.VMEM((N_L, H), jnp.float32),
            pltpu.VMEM((CHUNK, CHUNK), jnp.float32),
            pltpu.VMEM((R, QB + N_L), jnp.float32),
            pltpu.VMEM((QB, R), jnp.float32),
        ],
    )(qtr, llms_embedding, role_embT, W_qtr, W_l,
      b_qtlr2, W_roleT, b_roleT, rand_u2, active)
    return out[0].reshape(N_Q, N_L), out[1]


def kernel(queries, tasks, llms_embedding, llms_num, reasonings, role_emb,
           W_qtlr, b_qtlr, W_role, b_role, rand_u):
    qtr = jnp.concatenate([queries, tasks, reasonings], axis=1)
    W_qtr = jnp.concatenate([W_qtlr[0:D], W_qtlr[D:2 * D],
                             W_qtlr[3 * D:4 * D]], axis=0)
    W_l = W_qtlr[2 * D:3 * D]
    role_embT = role_emb.T
    W_roleT = W_role.T
    b_qtlr2 = b_qtlr.reshape(1, H)
    b_roleT = b_role.reshape(H, 1)
    rand_u2 = rand_u.reshape(N_Q * N_L, 1)
    active = (llms_num > 0).astype(jnp.float32).reshape(N_Q * N_L, 1)
    return _run(qtr, llms_embedding, role_embT, W_qtr, W_l,
                b_qtlr2, W_roleT, b_roleT, rand_u2, active)


# speedup vs baseline: 5.0035x; 1.4770x over previous
import jax
import jax.numpy as jnp
from jax import lax
from jax.experimental import pallas as pl
from jax.experimental.pallas import tpu as pltpu

N_Q = 1024
N_L = 16
D = 384
H = 64
N_ROLES = 1024
QB = 32
CHUNK = 128
NCH = N_ROLES // CHUNK
R = QB * N_L
GRID = N_Q // QB


def _body(qtrT_ref, llmsT_ref, remb_ref, wrole_ref, brole_ref,
          wqtrT_ref, wlT_ref, bqT_ref, u_ref, act_ref,
          sel_ref, lp_out_ref,
          rt_ref, lpT_ref, g2T_ref, gqT_ref, tri_ref):
    f32 = jnp.float32

    @pl.when(pl.program_id(0) == 0)
    def _init():
        row = lax.broadcasted_iota(jnp.int32, (CHUNK, CHUNK), 0)
        col = lax.broadcasted_iota(jnp.int32, (CHUNK, CHUNK), 1)
        tri_ref[...] = (col <= row).astype(f32)
        kk = lax.broadcasted_iota(jnp.int32, (QB + N_L, R), 0)
        rr = lax.broadcasted_iota(jnp.int32, (QB + N_L, R), 1)
        g2T_ref[...] = jnp.where(kk < QB, (kk == rr // N_L).astype(f32),
                                 (kk - QB == rr % N_L).astype(f32))
        rc = lax.broadcasted_iota(jnp.int32, (R, QB), 0)
        qq = lax.broadcasted_iota(jnp.int32, (R, QB), 1)
        gqT_ref[...] = (qq == rc // N_L).astype(f32)
        rt_un = (jnp.dot(remb_ref[...], wrole_ref[...],
                         preferred_element_type=f32) + brole_ref[...])
        nn = jnp.dot(rt_un * rt_un, jnp.ones((H, 1), f32),
                     preferred_element_type=f32)
        rt_ref[...] = rt_un * (1.0 / jnp.maximum(jnp.sqrt(nn), 1e-12))
        lpT_ref[...] = jnp.dot(wlT_ref[...], llmsT_ref[...],
                               preferred_element_type=f32)

    sT = (jnp.dot(wqtrT_ref[...], qtrT_ref[0],
                  preferred_element_type=f32) + bqT_ref[...])

    slpT = jnp.concatenate([sT, lpT_ref[...]], axis=1)
    g2T = g2T_ref[...]

    e2T = jnp.dot(slpT, g2T, preferred_element_type=f32)
    n2 = jnp.dot(jnp.ones((1, H), f32), e2T * e2T,
                 preferred_element_type=f32)
    inv = 1.0 / jnp.maximum(jnp.sqrt(n2), 1e-12)

    srlrT = jnp.dot(rt_ref[...], slpT, preferred_element_type=f32)
    eT = jnp.exp(jnp.dot(srlrT, g2T, preferred_element_type=f32) * inv)

    tri = tri_ref[...]
    fs = [jnp.dot(tri, eT[c * CHUNK:(c + 1) * CHUNK, :],
                  preferred_element_type=f32) for c in range(NCH)]
    carries = [jnp.zeros((1, R), f32)]
    for c in range(NCH):
        carries.append(carries[c] + fs[c][CHUNK - 1:CHUNK, :])
    z = carries[NCH]
    thr = u_ref[...] * z

    cntv = jnp.zeros((CHUNK, R), f32)
    for c in range(NCH):
        cntv = cntv + (fs[c] <= thr - carries[c]).astype(f32)
    cnt = jnp.dot(jnp.ones((1, CHUNK), f32), cntv,
                  preferred_element_type=f32).astype(jnp.int32)
    sel = jnp.where(cnt >= N_ROLES, 0, cnt)

    iota = lax.broadcasted_iota(jnp.int32, (N_ROLES, R), 0)
    masked = jnp.where(iota == sel, eT, 0.0)
    sel_e = jnp.dot(jnp.ones((1, N_ROLES), f32), masked,
                    preferred_element_type=f32)

    logterm = (jnp.log(sel_e) - jnp.log(z)) * act_ref[...]

    sel_ref[...] = sel[None]
    lp_out_ref[...] = jnp.dot(logterm, gqT_ref[...],
                              preferred_element_type=f32)[None]


@jax.jit
def _run(qtrT, llmsT, role_emb, W_role, b_role2, wqtrT, wlT, bqT,
         uT, actT):
    rep = lambda shape: pl.BlockSpec(shape, lambda i: (0,) * len(shape))
    out = pl.pallas_call(
        _body,
        grid=(GRID,),
        in_specs=[
            pl.BlockSpec((1, 3 * D, QB), lambda i: (i, 0, 0)),
            rep((D, N_L)), rep((N_ROLES, D)), rep((D, H)), rep((1, H)),
            rep((H, 3 * D)), rep((H, D)), rep((H, 1)),
            pl.BlockSpec((1, R), lambda i: (0, i)),
            pl.BlockSpec((1, R), lambda i: (0, i)),
        ],
        out_specs=[
            pl.BlockSpec((1, 1, R), lambda i: (i, 0, 0)),
            pl.BlockSpec((1, 1, QB), lambda i: (i, 0, 0)),
        ],
        out_shape=[
            jax.ShapeDtypeStruct((GRID, 1, R), jnp.int32),
            jax.ShapeDtypeStruct((GRID, 1, QB), jnp.float32),
        ],
        scratch_shapes=[
            pltpu.VMEM((N_ROLES, H), jnp.float32),
            pltpu.VMEM((H, N_L), jnp.float32),
            pltpu.VMEM((QB + N_L, R), jnp.float32),
            pltpu.VMEM((R, QB), jnp.float32),
            pltpu.VMEM((CHUNK, CHUNK), jnp.float32),
        ],
    )(qtrT, llmsT, role_emb, W_role, b_role2, wqtrT, wlT, bqT, uT, actT)
    return out[0].reshape(N_Q, N_L), out[1].reshape(N_Q, 1)


def kernel(queries, tasks, llms_embedding, llms_num, reasonings, role_emb,
           W_qtlr, b_qtlr, W_role, b_role, rand_u):
    qtrT = jnp.concatenate([queries, tasks, reasonings], axis=1).T
    qtrT = qtrT.reshape(3 * D, GRID, QB).transpose(1, 0, 2)
    wqtrT = jnp.concatenate([W_qtlr[0:D], W_qtlr[D:2 * D],
                             W_qtlr[3 * D:4 * D]], axis=0).T
    wlT = W_qtlr[2 * D:3 * D].T
    llmsT = llms_embedding.T
    b_role2 = b_role.reshape(1, H)
    bqT = b_qtlr.reshape(H, 1)
    uT = rand_u.reshape(1, N_Q * N_L)
    actT = (llms_num > 0).astype(jnp.float32).reshape(1, N_Q * N_L)
    return _run(qtrT, llmsT, role_emb, W_role, b_role2, wqtrT, wlT, bqT,
                uT, actT)
